# baseline (speedup 1.0000x reference)
import functools

import jax
import jax.numpy as jnp
from jax import lax
from jax.experimental import pallas as pl
from jax.experimental.pallas import tpu as pltpu
from jax.experimental.pallas import tpu_sc as plsc

N = 10000
E = 320000
D = 128
N_CLS = 47
ROWS = 512
GRID = (N + ROWS - 1) // ROWS

NC = 2
NS = 16
NW = NC * NS
NPW = 320
NPAD = NW * NPW
CAP = 12288
CHUNK_E = 2560
NCH_A = E // CHUNK_E
KG = 128

_SC_MESH = plsc.VectorSubcoreMesh(core_axis_name="c", subcore_axis_name="s")


@functools.partial(
    pl.kernel,
    out_type=[
        jax.ShapeDtypeStruct((NW, CAP), jnp.int32),
        jax.ShapeDtypeStruct((NW, CAP), jnp.int32),
        jax.ShapeDtypeStruct((NW, 128), jnp.int32),
    ],
    mesh=_SC_MESH,
    compiler_params=pltpu.CompilerParams(needs_layout_passes=False),
    scratch_types=[
        pltpu.VMEM((CAP,), jnp.int32),
        pltpu.VMEM((CAP,), jnp.int32),
        pltpu.VMEM((2, CHUNK_E), jnp.int32),
        pltpu.VMEM((2, CHUNK_E), jnp.int32),
        pltpu.VMEM((CAP,), jnp.int32),
        pltpu.VMEM((CAP,), jnp.int32),
        pltpu.VMEM((336,), jnp.int32),
        pltpu.VMEM((336,), jnp.int32),
        pltpu.VMEM((336,), jnp.int32),
        pltpu.VMEM((128,), jnp.int32),
        pltpu.SemaphoreType.DMA((2,)),
        pltpu.SemaphoreType.DMA((2,)),
    ],
)
def _sc_bin_edges(src_hbm, dst_hbm, csrc_hbm, cldst_hbm, cnt_hbm,
                  rsrc_v, rldst_v, dbuf, sbuf, csrc_v, cldst_v,
                  deg_v, offs_v, fill_v, cnt_v, dsem, ssem):
    wid = lax.axis_index("s") * NC + lax.axis_index("c")
    lo = wid * NPW
    hi = lo + NPW
    HALF = CAP // 2
    DUMMY = NPW * 16

    zeros16 = jnp.zeros((16,), jnp.int32)
    dummy_key16 = jnp.full((16,), NPW, jnp.int32)
    dummy_off16 = jnp.full((16,), DUMMY, jnp.int32)

    def init_body(i, _):
        rsrc_v[pl.ds(i * 16, 16)] = zeros16
        rldst_v[pl.ds(i * 16, 16)] = dummy_key16
        csrc_v[pl.ds(i * 16, 16)] = zeros16
        cldst_v[pl.ds(i * 16, 16)] = dummy_off16
        return 0
    lax.fori_loop(0, CAP // 16, init_body, 0)
    for i in range(336 // 16):
        deg_v[pl.ds(i * 16, 16)] = zeros16

    def issue(c, b):
        pltpu.async_copy(dst_hbm.at[pl.ds(c * CHUNK_E, CHUNK_E)], dbuf.at[b], dsem.at[b])
        pltpu.async_copy(src_hbm.at[pl.ds(c * CHUNK_E, CHUNK_E)], sbuf.at[b], ssem.at[b])

    issue(0, 0)

    def chunk_body(c, cnts):
        cnt_a, cnt_b = cnts
        b = lax.rem(c, 2)

        @pl.when(c + 1 < NCH_A)
        def _():
            issue(c + 1, 1 - b)

        pltpu.make_async_copy(dst_hbm.at[pl.ds(c * CHUNK_E, CHUNK_E)], dbuf.at[b], dsem.at[b]).wait()
        pltpu.make_async_copy(src_hbm.at[pl.ds(c * CHUNK_E, CHUNK_E)], sbuf.at[b], ssem.at[b]).wait()

        G2 = CHUNK_E // 32

        def group_body(g, cnts):
            cnt_a, cnt_b = cnts
            dva = dbuf[b, pl.ds(g * 16, 16)]
            sva = sbuf[b, pl.ds(g * 16, 16)]
            dvb = dbuf[b, pl.ds((G2 + g) * 16, 16)]
            svb = sbuf[b, pl.ds((G2 + g) * 16, 16)]
            ma = (dva >= lo) & (dva < hi)
            mb = (dvb >= lo) & (dvb < hi)
            inc_a = plsc.all_reduce_population_count(ma)[0]
            inc_b = plsc.all_reduce_population_count(mb)[0]
            plsc.store_compressed(rsrc_v.at[pl.ds(cnt_a, 16)], sva, mask=ma)
            plsc.store_compressed(rldst_v.at[pl.ds(cnt_a, 16)], dva - lo, mask=ma)
            plsc.store_compressed(rsrc_v.at[pl.ds(HALF + cnt_b, 16)], svb, mask=mb)
            plsc.store_compressed(rldst_v.at[pl.ds(HALF + cnt_b, 16)], dvb - lo, mask=mb)
            return (jnp.minimum(cnt_a + inc_a, HALF - 16),
                    jnp.minimum(cnt_b + inc_b, HALF - 16))

        return lax.fori_loop(0, G2, group_body, (cnt_a, cnt_b))

    cnt_a, cnt_b = lax.fori_loop(0, NCH_A, chunk_body,
                                 (jnp.int32(0), jnp.int32(0)))
    cnt = cnt_a + cnt_b

    fo = plsc.scan_count(zeros16)[0][0]

    ng_a = (cnt_a + 15) // 16
    ng_b = (cnt_b + 15) // 16

    def hist_body(g, base):
        keys = rldst_v[pl.ds(base + g * 16, 16)]
        occ, last = plsc.scan_count(keys)
        cur = plsc.load_gather(deg_v, [keys])
        plsc.store_scatter(deg_v, [keys], cur + occ - fo + 1, mask=last)
        return base
    lax.fori_loop(0, ng_a, hist_body, jnp.int32(0))
    lax.fori_loop(0, ng_b, hist_body, jnp.int32(HALF))

    def scan_body(g, carry):
        v = deg_v[pl.ds(g * 16, 16)]
        inc = plsc.cumsum(v)
        exc = inc - v + carry
        offs_v[pl.ds(g * 16, 16)] = exc
        fill_v[pl.ds(g * 16, 16)] = exc
        return carry + inc[15]
    lax.fori_loop(0, 336 // 16, scan_body, jnp.int32(0))

    def place_body(g, base):
        keys = rldst_v[pl.ds(base + g * 16, 16)]
        srcv = rsrc_v[pl.ds(base + g * 16, 16)]
        occ, last = plsc.scan_count(keys)
        fbase = plsc.load_gather(fill_v, [keys])
        pos = fbase + occ - fo
        plsc.store_scatter(csrc_v, [pos], srcv)
        plsc.store_scatter(cldst_v, [pos], keys * 16)
        plsc.store_scatter(fill_v, [keys], pos + 1, mask=last)
        return base
    lax.fori_loop(0, ng_a, place_body, jnp.int32(0))
    lax.fori_loop(0, ng_b, place_body, jnp.int32(HALF))

    def cnt_store(i, _):
        cnt_v[pl.ds(i * 16, 16)] = jnp.full((16,), 1, jnp.int32) * cnt
        return 0
    lax.fori_loop(0, 8, cnt_store, 0)
    pltpu.sync_copy(csrc_v, csrc_hbm.at[wid])
    pltpu.sync_copy(cldst_v, cldst_hbm.at[wid])
    pltpu.sync_copy(cnt_v, cnt_hbm.at[wid])


SLICE_W = NPW * 16
SLAB = (NPW + 1) * 16


@functools.partial(
    pl.kernel,
    out_type=[jax.ShapeDtypeStruct((NW * SLICE_W,), jnp.float32) for _ in range(8)],
    mesh=_SC_MESH,
    compiler_params=pltpu.CompilerParams(needs_layout_passes=False),
    scratch_types=[
        pltpu.VMEM((CAP,), jnp.int32),
        pltpu.VMEM((CAP,), jnp.int32),
        pltpu.VMEM((128,), jnp.int32),
        [pltpu.VMEM((SLAB,), jnp.float32) for _ in range(8)],
        pltpu.VMEM((3, KG, D), jnp.float32),
        pltpu.SemaphoreType.DMA((3,)),
    ],
)
def _sc_seg_max(hp_hbm, bsrc_hbm, bldst_hbm, cnt_hbm, *refs):
    out_hbm = refs[:8]
    bsrc_v, bldst_v, cnt_v, aggs, gbuf, gsem = refs[8:]
    wid = lax.axis_index("s") * NC + lax.axis_index("c")

    pltpu.sync_copy(bsrc_hbm.at[wid], bsrc_v)
    pltpu.sync_copy(bldst_hbm.at[wid], bldst_v)
    pltpu.sync_copy(cnt_hbm.at[wid], cnt_v)
    cnt = cnt_v[pl.ds(0, 16)][0]

    zf = jnp.zeros((16,), jnp.float32)

    def zero_body(i, _):
        for t in range(8):
            aggs[t][pl.ds(i * 16, 16)] = zf
        return 0
    lax.fori_loop(0, SLAB // 16, zero_body, 0)

    nchunks = (cnt + (KG - 1)) // KG

    def issue(c, b):
        pltpu.async_copy(hp_hbm.at[bsrc_v.at[pl.ds(c * KG, KG)]], gbuf.at[b], gsem.at[b])

    @pl.when(nchunks > 0)
    def _():
        issue(0, 0)

    @pl.when(nchunks > 1)
    def _():
        issue(1, 1)

    def chunk_body(c, carry):
        b = lax.rem(c, 3)

        @pl.when(c + 2 < nchunks)
        def _():
            issue(c + 2, lax.rem(c + 2, 3))

        pltpu.make_async_copy(hp_hbm.at[bsrc_v.at[pl.ds(c * KG, KG)]], gbuf.at[b], gsem.at[b]).wait()

        def edge_body(g, carry):
            o_prev, acc = carry
            sv = bldst_v[pl.ds(c * KG + g * 16, 16)]
            for l in range(16):
                o = sv[l]
                j = g * 16 + l
                first = (o != o_prev).astype(jnp.int32)
                m = (jnp.full((16,), 1, jnp.int32) * first) != 0
                rs = [gbuf[b, j, pl.ds(t * 16, 16)] for t in range(8)]
                for t in range(8):
                    plsc.store_compressed(aggs[t].at[pl.ds(o_prev, 16)], acc[t], mask=m)
                acc = tuple(jnp.where(m, rs[t], jnp.maximum(acc[t], rs[t]))
                            for t in range(8))
                o_prev = o
            return o_prev, acc

        return lax.fori_loop(0, KG // 16, edge_body, carry)

    acc0 = tuple(jnp.zeros((16,), jnp.float32) for _ in range(8))
    o_prev, acc = lax.fori_loop(0, nchunks, chunk_body,
                                (jnp.int32(NPW * 16), acc0))
    for t in range(8):
        aggs[t][pl.ds(o_prev, 16)] = acc[t]

    for t in range(8):
        pltpu.sync_copy(aggs[t].at[pl.ds(0, SLICE_W)], out_hbm[t].at[pl.ds(wid * SLICE_W, SLICE_W)])


def _t0_body(x_ref, wp_ref, bp_ref, ws_ref, bs_ref, hp_ref, s_ref):
    x = x_ref[...]
    hp = jnp.dot(x, wp_ref[...], preferred_element_type=jnp.float32) + bp_ref[...]
    hp_ref[...] = jnp.maximum(hp, 0.0)
    s_ref[...] = jnp.dot(x, ws_ref[...], preferred_element_type=jnp.float32) + bs_ref[...]


def _tc_pre(x, Wp, bp, Ws, bs):
    dh = Ws.shape[1]
    return pl.pallas_call(
        _t0_body,
        grid=(GRID,),
        in_specs=[
            pl.BlockSpec((ROWS, D), lambda i: (i, 0)),
            pl.BlockSpec((D, D), lambda i: (0, 0)),
            pl.BlockSpec((1, D), lambda i: (0, 0)),
            pl.BlockSpec((D, dh), lambda i: (0, 0)),
            pl.BlockSpec((1, dh), lambda i: (0, 0)),
        ],
        out_specs=[
            pl.BlockSpec((ROWS, D), lambda i: (i, 0)),
            pl.BlockSpec((ROWS, dh), lambda i: (i, 0)),
        ],
        out_shape=[
            jax.ShapeDtypeStruct((N, D), jnp.float32),
            jax.ShapeDtypeStruct((N, dh), jnp.float32),
        ],
    )(x, Wp, bp.reshape(1, D), Ws, bs.reshape(1, dh))


def _tmid_body(s_ref, agg_ref, wn_ref, wp_ref, bp_ref, ws_ref, bs_ref, hp_ref, s2_ref):
    x = jnp.maximum(
        s_ref[...]
        + jnp.dot(agg_ref[...], wn_ref[...], preferred_element_type=jnp.float32),
        0.0,
    )
    hp = jnp.dot(x, wp_ref[...], preferred_element_type=jnp.float32) + bp_ref[...]
    hp_ref[...] = jnp.maximum(hp, 0.0)
    s2_ref[...] = jnp.dot(x, ws_ref[...], preferred_element_type=jnp.float32) + bs_ref[...]


def _tc_mid(s, agg, Wn, Wp, bp, Ws, bs):
    dh = Ws.shape[1]
    return pl.pallas_call(
        _tmid_body,
        grid=(GRID,),
        in_specs=[
            pl.BlockSpec((ROWS, D), lambda i: (i, 0)),
            pl.BlockSpec((ROWS, D), lambda i: (i, 0)),
            pl.BlockSpec((D, D), lambda i: (0, 0)),
            pl.BlockSpec((D, D), lambda i: (0, 0)),
            pl.BlockSpec((1, D), lambda i: (0, 0)),
            pl.BlockSpec((D, dh), lambda i: (0, 0)),
            pl.BlockSpec((1, dh), lambda i: (0, 0)),
        ],
        out_specs=[
            pl.BlockSpec((ROWS, D), lambda i: (i, 0)),
            pl.BlockSpec((ROWS, dh), lambda i: (i, 0)),
        ],
        out_shape=[
            jax.ShapeDtypeStruct((N, D), jnp.float32),
            jax.ShapeDtypeStruct((N, dh), jnp.float32),
        ],
    )(s, agg, Wn, Wp, bp.reshape(1, D), Ws, bs.reshape(1, dh))


def _tfinal_body(s_ref, agg_ref, wn_ref, out_ref):
    h = jnp.maximum(
        s_ref[...]
        + jnp.dot(agg_ref[...], wn_ref[...], preferred_element_type=jnp.float32),
        0.0,
    )
    m = jnp.max(h, axis=-1, keepdims=True)
    lse = jnp.log(jnp.sum(jnp.exp(h - m), axis=-1, keepdims=True))
    out_ref[...] = h - m - lse


def _tc_final(s, agg, Wn):
    return pl.pallas_call(
        _tfinal_body,
        grid=(GRID,),
        in_specs=[
            pl.BlockSpec((ROWS, N_CLS), lambda i: (i, 0)),
            pl.BlockSpec((ROWS, D), lambda i: (i, 0)),
            pl.BlockSpec((D, N_CLS), lambda i: (0, 0)),
        ],
        out_specs=pl.BlockSpec((ROWS, N_CLS), lambda i: (i, 0)),
        out_shape=jax.ShapeDtypeStruct((N, N_CLS), jnp.float32),
    )(s, agg, Wn)


def _segmax_sc(hp, bsrc, bldst, cnt):
    slabs = _sc_seg_max(hp, bsrc, bldst, cnt)
    return jnp.concatenate([s.reshape(NPAD, 16) for s in slabs], axis=1)[:N]


def kernel(features, edge_index, Wp0, bp0, Ws0, bs0, Wn0, Wp1, bp1, Ws1, bs1, Wn1, Wp2, bp2, Ws2, bs2, Wn2):
    src = edge_index[0]
    dst = edge_index[1]
    bsrc, bldst, cnt = _sc_bin_edges(src, dst)
    hp0, s0 = _tc_pre(features, Wp0, bp0, Ws0, bs0)
    agg0 = _segmax_sc(hp0, bsrc, bldst, cnt)
    hp1, s1 = _tc_mid(s0, agg0, Wn0, Wp1, bp1, Ws1, bs1)
    agg1 = _segmax_sc(hp1, bsrc, bldst, cnt)
    hp2, s2 = _tc_mid(s1, agg1, Wn1, Wp2, bp2, Ws2, bs2)
    agg2 = _segmax_sc(hp2, bsrc, bldst, cnt)
    return _tc_final(s2, agg2, Wn2)


# speedup vs baseline: 1.1741x; 1.1741x over previous
import functools

import jax
import jax.numpy as jnp
from jax import lax
from jax.experimental import pallas as pl
from jax.experimental.pallas import tpu as pltpu
from jax.experimental.pallas import tpu_sc as plsc

N = 10000
E = 320000
D = 128
N_CLS = 47
ROWS = 512
GRID = (N + ROWS - 1) // ROWS
NROW_PAD = GRID * ROWS

NC = 2
NS = 16
NW = NC * NS
NPW = 320
NPAD = NW * NPW
CAP = 12288
CHUNK_E = 2560
NCH_A = E // CHUNK_E
KG = 128

_SC_MESH = plsc.VectorSubcoreMesh(core_axis_name="c", subcore_axis_name="s")


@functools.partial(
    pl.kernel,
    out_type=[
        jax.ShapeDtypeStruct((NW, CAP), jnp.int32),
        jax.ShapeDtypeStruct((NW, CAP), jnp.int32),
        jax.ShapeDtypeStruct((NW, 128), jnp.int32),
    ],
    mesh=_SC_MESH,
    compiler_params=pltpu.CompilerParams(needs_layout_passes=False),
    scratch_types=[
        pltpu.VMEM((CAP,), jnp.int32),
        pltpu.VMEM((CAP,), jnp.int32),
        pltpu.VMEM((2, CHUNK_E), jnp.int32),
        pltpu.VMEM((2, CHUNK_E), jnp.int32),
        pltpu.VMEM((CAP,), jnp.int32),
        pltpu.VMEM((CAP,), jnp.int32),
        pltpu.VMEM((336,), jnp.int32),
        pltpu.VMEM((336,), jnp.int32),
        pltpu.VMEM((336,), jnp.int32),
        pltpu.VMEM((128,), jnp.int32),
        pltpu.SemaphoreType.DMA((2,)),
        pltpu.SemaphoreType.DMA((2,)),
    ],
)
def _sc_bin_edges(src_hbm, dst_hbm, csrc_hbm, cldst_hbm, cnt_hbm,
                  rsrc_v, rldst_v, dbuf, sbuf, csrc_v, cldst_v,
                  deg_v, offs_v, fill_v, cnt_v, dsem, ssem):
    wid = lax.axis_index("s") * NC + lax.axis_index("c")
    lo = wid * NPW
    hi = lo + NPW
    HALF = CAP // 2
    DUMMY = NPW * 16

    zeros16 = jnp.zeros((16,), jnp.int32)
    dummy_key16 = jnp.full((16,), NPW, jnp.int32)
    dummy_off16 = jnp.full((16,), DUMMY, jnp.int32)

    def init_body(i, _):
        rsrc_v[pl.ds(i * 16, 16)] = zeros16
        rldst_v[pl.ds(i * 16, 16)] = dummy_key16
        csrc_v[pl.ds(i * 16, 16)] = zeros16
        cldst_v[pl.ds(i * 16, 16)] = dummy_off16
        return 0
    lax.fori_loop(0, CAP // 16, init_body, 0)
    for i in range(336 // 16):
        deg_v[pl.ds(i * 16, 16)] = zeros16

    def issue(c, b):
        pltpu.async_copy(dst_hbm.at[pl.ds(c * CHUNK_E, CHUNK_E)], dbuf.at[b], dsem.at[b])
        pltpu.async_copy(src_hbm.at[pl.ds(c * CHUNK_E, CHUNK_E)], sbuf.at[b], ssem.at[b])

    issue(0, 0)

    def chunk_body(c, cnts):
        cnt_a, cnt_b = cnts
        b = lax.rem(c, 2)

        @pl.when(c + 1 < NCH_A)
        def _():
            issue(c + 1, 1 - b)

        pltpu.make_async_copy(dst_hbm.at[pl.ds(c * CHUNK_E, CHUNK_E)], dbuf.at[b], dsem.at[b]).wait()
        pltpu.make_async_copy(src_hbm.at[pl.ds(c * CHUNK_E, CHUNK_E)], sbuf.at[b], ssem.at[b]).wait()

        G2 = CHUNK_E // 32

        def group_body(g, cnts):
            cnt_a, cnt_b = cnts
            dva = dbuf[b, pl.ds(g * 16, 16)]
            sva = sbuf[b, pl.ds(g * 16, 16)]
            dvb = dbuf[b, pl.ds((G2 + g) * 16, 16)]
            svb = sbuf[b, pl.ds((G2 + g) * 16, 16)]
            ma = (dva >= lo) & (dva < hi)
            mb = (dvb >= lo) & (dvb < hi)
            inc_a = plsc.all_reduce_population_count(ma)[0]
            inc_b = plsc.all_reduce_population_count(mb)[0]
            plsc.store_compressed(rsrc_v.at[pl.ds(cnt_a, 16)], sva, mask=ma)
            plsc.store_compressed(rldst_v.at[pl.ds(cnt_a, 16)], dva - lo, mask=ma)
            plsc.store_compressed(rsrc_v.at[pl.ds(HALF + cnt_b, 16)], svb, mask=mb)
            plsc.store_compressed(rldst_v.at[pl.ds(HALF + cnt_b, 16)], dvb - lo, mask=mb)
            return (jnp.minimum(cnt_a + inc_a, HALF - 16),
                    jnp.minimum(cnt_b + inc_b, HALF - 16))

        return lax.fori_loop(0, G2, group_body, (cnt_a, cnt_b))

    cnt_a, cnt_b = lax.fori_loop(0, NCH_A, chunk_body,
                                 (jnp.int32(0), jnp.int32(0)))
    cnt = cnt_a + cnt_b

    fo = plsc.scan_count(zeros16)[0][0]

    ng_a = (cnt_a + 15) // 16
    ng_b = (cnt_b + 15) // 16

    def hist_body(g, base):
        keys = rldst_v[pl.ds(base + g * 16, 16)]
        occ, last = plsc.scan_count(keys)
        cur = plsc.load_gather(deg_v, [keys])
        plsc.store_scatter(deg_v, [keys], cur + occ - fo + 1, mask=last)
        return base
    lax.fori_loop(0, ng_a, hist_body, jnp.int32(0))
    lax.fori_loop(0, ng_b, hist_body, jnp.int32(HALF))

    def scan_body(g, carry):
        v = deg_v[pl.ds(g * 16, 16)]
        inc = plsc.cumsum(v)
        exc = inc - v + carry
        offs_v[pl.ds(g * 16, 16)] = exc
        fill_v[pl.ds(g * 16, 16)] = exc
        return carry + inc[15]
    lax.fori_loop(0, 336 // 16, scan_body, jnp.int32(0))

    def place_body(g, base):
        keys = rldst_v[pl.ds(base + g * 16, 16)]
        srcv = rsrc_v[pl.ds(base + g * 16, 16)]
        occ, last = plsc.scan_count(keys)
        fbase = plsc.load_gather(fill_v, [keys])
        pos = fbase + occ - fo
        plsc.store_scatter(csrc_v, [pos], srcv)
        plsc.store_scatter(cldst_v, [pos], keys * 16)
        plsc.store_scatter(fill_v, [keys], pos + 1, mask=last)
        return base
    lax.fori_loop(0, ng_a, place_body, jnp.int32(0))
    lax.fori_loop(0, ng_b, place_body, jnp.int32(HALF))

    def cnt_store(i, _):
        cnt_v[pl.ds(i * 16, 16)] = jnp.full((16,), 1, jnp.int32) * cnt
        return 0
    lax.fori_loop(0, 8, cnt_store, 0)
    pltpu.sync_copy(csrc_v, csrc_hbm.at[wid])
    pltpu.sync_copy(cldst_v, cldst_hbm.at[wid])
    pltpu.sync_copy(cnt_v, cnt_hbm.at[wid])


SLICE_W = NPW * 16
SLAB = (NPW + 1) * 16


@functools.partial(
    pl.kernel,
    out_type=[jax.ShapeDtypeStruct((NW * SLICE_W,), jnp.float32) for _ in range(8)],
    mesh=_SC_MESH,
    compiler_params=pltpu.CompilerParams(needs_layout_passes=False),
    scratch_types=[
        pltpu.VMEM((CAP,), jnp.int32),
        pltpu.VMEM((CAP,), jnp.int32),
        pltpu.VMEM((128,), jnp.int32),
        [pltpu.VMEM((SLAB,), jnp.float32) for _ in range(8)],
        pltpu.VMEM((3, KG, D), jnp.float32),
        pltpu.SemaphoreType.DMA((3,)),
    ],
)
def _sc_seg_max(hp_hbm, bsrc_hbm, bldst_hbm, cnt_hbm, *refs):
    out_hbm = refs[:8]
    bsrc_v, bldst_v, cnt_v, aggs, gbuf, gsem = refs[8:]
    wid = lax.axis_index("s") * NC + lax.axis_index("c")

    pltpu.sync_copy(bsrc_hbm.at[wid], bsrc_v)
    pltpu.sync_copy(bldst_hbm.at[wid], bldst_v)
    pltpu.sync_copy(cnt_hbm.at[wid], cnt_v)
    cnt = cnt_v[pl.ds(0, 16)][0]

    zf = jnp.zeros((16,), jnp.float32)

    def zero_body(i, _):
        for t in range(8):
            aggs[t][pl.ds(i * 16, 16)] = zf
        return 0
    lax.fori_loop(0, SLAB // 16, zero_body, 0)

    nchunks = (cnt + (KG - 1)) // KG

    def issue(c, b):
        pltpu.async_copy(hp_hbm.at[bsrc_v.at[pl.ds(c * KG, KG)]], gbuf.at[b], gsem.at[b])

    @pl.when(nchunks > 0)
    def _():
        issue(0, 0)

    @pl.when(nchunks > 1)
    def _():
        issue(1, 1)

    def chunk_body(c, carry):
        b = lax.rem(c, 3)

        @pl.when(c + 2 < nchunks)
        def _():
            issue(c + 2, lax.rem(c + 2, 3))

        pltpu.make_async_copy(hp_hbm.at[bsrc_v.at[pl.ds(c * KG, KG)]], gbuf.at[b], gsem.at[b]).wait()

        def edge_body(g, carry):
            o_prev, acc = carry
            sv = bldst_v[pl.ds(c * KG + g * 16, 16)]
            for l in range(16):
                o = sv[l]
                j = g * 16 + l
                m = jnp.full((16,), o == o_prev)
                rs = [gbuf[b, j, pl.ds(t * 16, 16)] for t in range(8)]
                acc = tuple(jnp.maximum(jnp.where(m, acc[t], zf), rs[t])
                            for t in range(8))
                for t in range(8):
                    aggs[t][pl.ds(o, 16)] = acc[t]
                o_prev = o
            return o_prev, acc

        return lax.fori_loop(0, KG // 16, edge_body, carry)

    acc0 = tuple(jnp.zeros((16,), jnp.float32) for _ in range(8))
    lax.fori_loop(0, nchunks, chunk_body, (jnp.int32(NPW * 16), acc0))

    for t in range(8):
        pltpu.sync_copy(aggs[t].at[pl.ds(0, SLICE_W)], out_hbm[t].at[pl.ds(wid * SLICE_W, SLICE_W)])


def _t0_body(x_ref, wp_ref, bp_ref, ws_ref, bs_ref, hp_ref, s_ref):
    x = x_ref[...]
    hp = jnp.dot(x, wp_ref[...], preferred_element_type=jnp.float32) + bp_ref[...]
    hp_ref[...] = jnp.maximum(hp, 0.0)
    s_ref[...] = jnp.dot(x, ws_ref[...], preferred_element_type=jnp.float32) + bs_ref[...]


def _tc_pre(x, Wp, bp, Ws, bs):
    dh = Ws.shape[1]
    return pl.pallas_call(
        _t0_body,
        grid=(GRID,),
        in_specs=[
            pl.BlockSpec((ROWS, D), lambda i: (i, 0)),
            pl.BlockSpec((D, D), lambda i: (0, 0)),
            pl.BlockSpec((1, D), lambda i: (0, 0)),
            pl.BlockSpec((D, dh), lambda i: (0, 0)),
            pl.BlockSpec((1, dh), lambda i: (0, 0)),
        ],
        out_specs=[
            pl.BlockSpec((ROWS, D), lambda i: (i, 0)),
            pl.BlockSpec((ROWS, dh), lambda i: (i, 0)),
        ],
        out_shape=[
            jax.ShapeDtypeStruct((NROW_PAD, D), jnp.float32),
            jax.ShapeDtypeStruct((N, dh), jnp.float32),
        ],
    )(x, Wp, bp.reshape(1, D), Ws, bs.reshape(1, dh))


def _slab_spec():
    return pl.BlockSpec((ROWS, 16), lambda i: (i, 0))


def _unpack_agg(slab_refs):
    return jnp.concatenate([r[...] for r in slab_refs], axis=1)


def _tmid_body(s_ref, a0, a1, a2, a3, a4, a5, a6, a7, wn_ref, wp_ref, bp_ref, ws_ref, bs_ref, hp_ref, s2_ref):
    agg = _unpack_agg([a0, a1, a2, a3, a4, a5, a6, a7])
    x = jnp.maximum(
        s_ref[...]
        + jnp.dot(agg, wn_ref[...], preferred_element_type=jnp.float32),
        0.0,
    )
    hp = jnp.dot(x, wp_ref[...], preferred_element_type=jnp.float32) + bp_ref[...]
    hp_ref[...] = jnp.maximum(hp, 0.0)
    s2_ref[...] = jnp.dot(x, ws_ref[...], preferred_element_type=jnp.float32) + bs_ref[...]


def _tc_mid(s, agg, Wn, Wp, bp, Ws, bs):
    dh = Ws.shape[1]
    return pl.pallas_call(
        _tmid_body,
        grid=(GRID,),
        in_specs=[
            pl.BlockSpec((ROWS, D), lambda i: (i, 0)),
        ] + [_slab_spec() for _ in range(8)] + [
            pl.BlockSpec((D, D), lambda i: (0, 0)),
            pl.BlockSpec((D, D), lambda i: (0, 0)),
            pl.BlockSpec((1, D), lambda i: (0, 0)),
            pl.BlockSpec((D, dh), lambda i: (0, 0)),
            pl.BlockSpec((1, dh), lambda i: (0, 0)),
        ],
        out_specs=[
            pl.BlockSpec((ROWS, D), lambda i: (i, 0)),
            pl.BlockSpec((ROWS, dh), lambda i: (i, 0)),
        ],
        out_shape=[
            jax.ShapeDtypeStruct((NROW_PAD, D), jnp.float32),
            jax.ShapeDtypeStruct((N, dh), jnp.float32),
        ],
    )(s, *agg, Wn, Wp, bp.reshape(1, D), Ws, bs.reshape(1, dh))


def _tfinal_body(s_ref, a0, a1, a2, a3, a4, a5, a6, a7, wn_ref, out_ref):
    agg = _unpack_agg([a0, a1, a2, a3, a4, a5, a6, a7])
    h = jnp.maximum(
        s_ref[...]
        + jnp.dot(agg, wn_ref[...], preferred_element_type=jnp.float32),
        0.0,
    )
    m = jnp.max(h, axis=-1, keepdims=True)
    lse = jnp.log(jnp.sum(jnp.exp(h - m), axis=-1, keepdims=True))
    out_ref[...] = h - m - lse


def _tc_final(s, agg, Wn):
    return pl.pallas_call(
        _tfinal_body,
        grid=(GRID,),
        in_specs=[
            pl.BlockSpec((ROWS, N_CLS), lambda i: (i, 0)),
        ] + [_slab_spec() for _ in range(8)] + [
            pl.BlockSpec((D, N_CLS), lambda i: (0, 0)),
        ],
        out_specs=pl.BlockSpec((ROWS, N_CLS), lambda i: (i, 0)),
        out_shape=jax.ShapeDtypeStruct((N, N_CLS), jnp.float32),
    )(s, *agg, Wn)


def _segmax_sc(hp, bsrc, bldst, cnt):
    slabs = _sc_seg_max(hp, bsrc, bldst, cnt)
    return [s.reshape(NPAD, 16) for s in slabs]


def kernel(features, edge_index, Wp0, bp0, Ws0, bs0, Wn0, Wp1, bp1, Ws1, bs1, Wn1, Wp2, bp2, Ws2, bs2, Wn2):
    src = edge_index[0]
    dst = edge_index[1]
    bsrc, bldst, cnt = _sc_bin_edges(src, dst)
    hp0, s0 = _tc_pre(features, Wp0, bp0, Ws0, bs0)
    agg0 = _segmax_sc(hp0, bsrc, bldst, cnt)
    hp1, s1 = _tc_mid(s0, agg0, Wn0, Wp1, bp1, Ws1, bs1)
    agg1 = _segmax_sc(hp1, bsrc, bldst, cnt)
    hp2, s2 = _tc_mid(s1, agg1, Wn1, Wp2, bp2, Ws2, bs2)
    agg2 = _segmax_sc(hp2, bsrc, bldst, cnt)
    return _tc_final(s2, agg2, Wn2)
